# Initial kernel scaffold; baseline (speedup 1.0000x reference)
import jax
import jax.numpy as jnp
from jax.experimental import pallas as pl


def kernel(x, W_in, W_pos):
    raise NotImplementedError("write your pallas kernel here")


# speedup vs baseline: 2.7571x; 2.7571x over previous
import functools

import jax
import jax.numpy as jnp
from jax import lax
from jax.experimental import pallas as pl
from jax.experimental.pallas import tpu as pltpu
from jax.experimental.pallas import tpu_sc as plsc

NUM_EMB = 1000000
EMB_DIM = 64
MAX_LEN = 200
BATCH = 4096
SEQ = 200

NUM_CORES = 2
NUM_SUBCORES = 16
NUM_WORKERS = NUM_CORES * NUM_SUBCORES
SEQ_PER_W = BATCH // NUM_WORKERS
ROWS_PER_W = SEQ_PER_W * SEQ
LANES = 16
VECS_PER_ROW = EMB_DIM // LANES


def _lookup_kernel(w_hbm, x_hbm, pos_hbm, out_hbm,
                   idx_v, pos_v, buf0, buf1, sem0, sem1):
    wid = lax.axis_index("c") * NUM_SUBCORES + lax.axis_index("s")
    row_base = wid * ROWS_PER_W

    pltpu.sync_copy(x_hbm.at[pl.ds(row_base, ROWS_PER_W)], idx_v)
    pltpu.sync_copy(pos_hbm, pos_v)

    def issue(s, buf, sem):
        idx_slice = idx_v.at[pl.ds(s * SEQ, SEQ)]
        return pltpu.async_copy(w_hbm.at[idx_slice], buf, sem)

    def wait(buf, sem):
        pltpu.make_async_copy(w_hbm.at[idx_v.at[pl.ds(0, SEQ)]], buf, sem).wait()

    def add_pos_and_store(s, buf):
        def body(r, carry):
            for j in range(VECS_PER_ROW):
                plsc.addupdate(buf.at[r, pl.ds(j * LANES, LANES)],
                               pos_v[r, pl.ds(j * LANES, LANES)])
            return carry
        lax.fori_loop(0, SEQ, body, 0)
        pltpu.sync_copy(buf, out_hbm.at[pl.ds(row_base + s * SEQ, SEQ)])

    issue(0, buf0, sem0)

    def loop(g, carry):
        s0 = 2 * g
        s1 = s0 + 1
        issue(s1, buf1, sem1)
        wait(buf0, sem0)
        add_pos_and_store(s0, buf0)

        @pl.when(g < SEQ_PER_W // 2 - 1)
        def _():
            issue(s1 + 1, buf0, sem0)

        wait(buf1, sem1)
        add_pos_and_store(s1, buf1)
        return carry

    lax.fori_loop(0, SEQ_PER_W // 2, loop, 0)


@functools.partial(jax.jit, donate_argnums=())
def _lookup(W_in, x_flat, W_pos):
    mesh = plsc.VectorSubcoreMesh(core_axis_name="c", subcore_axis_name="s")
    f = functools.partial(
        pl.kernel,
        mesh=mesh,
        compiler_params=pltpu.CompilerParams(use_tc_tiling_on_sc=False),
        out_type=jax.ShapeDtypeStruct((BATCH * SEQ, EMB_DIM), jnp.float32),
        scratch_types=[
            pltpu.VMEM((ROWS_PER_W,), jnp.int32),
            pltpu.VMEM((MAX_LEN, EMB_DIM), jnp.float32),
            pltpu.VMEM((SEQ, EMB_DIM), jnp.float32),
            pltpu.VMEM((SEQ, EMB_DIM), jnp.float32),
            pltpu.SemaphoreType.DMA,
            pltpu.SemaphoreType.DMA,
        ],
    )(_lookup_kernel)
    return f(W_in, x_flat, W_pos)


def kernel(x, W_in, W_pos):
    x_flat = x.reshape(-1)
    out = _lookup(W_in, x_flat, W_pos)
    return out.reshape(BATCH, SEQ, EMB_DIM)
